# baseline (speedup 1.0000x reference)
import functools

import jax
import jax.numpy as jnp
from jax import lax
from jax.experimental import pallas as pl
from jax.experimental.pallas import tpu as pltpu
from jax.experimental.pallas import tpu_sc as plsc

VOCAB = 1000000
B = 16384
EMB = 64
INPUT_DIM = 2 * EMB
HID = 128
OUT = 64

NC = 2
NS = 16
NW = NC * NS
B_PER_W = B // NW
CHUNK = 128
NCHUNK = B_PER_W // CHUNK


def _sc_gather_body(emb_t_hbm, emb_l_hbm, idx_t_hbm, idx_l_hbm,
                    out_t_hbm, out_l_hbm,
                    idx_t_v, idx_l_v, rows_t_v, rows_l_v, sem):
  wid = lax.axis_index("s") * NC + lax.axis_index("c")
  base_chunk = wid * NCHUNK
  base_row = wid * B_PER_W

  pltpu.sync_copy(idx_t_hbm.at[pl.ds(base_chunk, NCHUNK)], idx_t_v)
  pltpu.sync_copy(idx_l_hbm.at[pl.ds(base_chunk, NCHUNK)], idx_l_v)

  copies = []
  for j in range(NCHUNK):
    copies.append(pltpu.async_copy(
        emb_t_hbm.at[idx_t_v.at[j]], rows_t_v.at[pl.ds(j * CHUNK, CHUNK)],
        sem))
    copies.append(pltpu.async_copy(
        emb_l_hbm.at[idx_l_v.at[j]], rows_l_v.at[pl.ds(j * CHUNK, CHUNK)],
        sem))
  for c in copies:
    c.wait()

  pltpu.sync_copy(rows_t_v, out_t_hbm.at[pl.ds(base_row, B_PER_W)])
  pltpu.sync_copy(rows_l_v, out_l_hbm.at[pl.ds(base_row, B_PER_W)])


@functools.cache
def _sc_gather():
  return pl.kernel(
      _sc_gather_body,
      out_type=(
          jax.ShapeDtypeStruct((B, EMB), jnp.float32),
          jax.ShapeDtypeStruct((B, EMB), jnp.float32),
      ),
      mesh=plsc.VectorSubcoreMesh(core_axis_name="c", subcore_axis_name="s"),
      compiler_params=pltpu.CompilerParams(use_tc_tiling_on_sc=False),
      scratch_types=[
          pltpu.VMEM((NCHUNK, CHUNK), jnp.int32),
          pltpu.VMEM((NCHUNK, CHUNK), jnp.int32),
          pltpu.VMEM((B_PER_W, EMB), jnp.float32),
          pltpu.VMEM((B_PER_W, EMB), jnp.float32),
          pltpu.SemaphoreType.DMA,
      ],
  )


BT = 2048


def _mlp_body(et_ref, el_ref, ln1w_ref, ln1b_ref, w1t_ref, b1_ref,
              w2t_ref, b2_ref, ln2w_ref, ln2b_ref, out_ref):
  x = jnp.concatenate([et_ref[...], el_ref[...]], axis=1)
  mu = jnp.mean(x, axis=1, keepdims=True)
  xc = x - mu
  var = jnp.mean(xc * xc, axis=1, keepdims=True)
  h = xc * jax.lax.rsqrt(var + 1e-5) * ln1w_ref[...] + ln1b_ref[...]
  h = jnp.dot(h, w1t_ref[...], preferred_element_type=jnp.float32)
  h = jnp.maximum(h + b1_ref[...], 0.0)
  y = jnp.dot(h, w2t_ref[...], preferred_element_type=jnp.float32)
  y = y + b2_ref[...]
  mu2 = jnp.mean(y, axis=1, keepdims=True)
  yc = y - mu2
  var2 = jnp.mean(yc * yc, axis=1, keepdims=True)
  out_ref[...] = yc * jax.lax.rsqrt(var2 + 1e-5) * ln2w_ref[...] + ln2b_ref[...]


def _mlp(et, el, ln1w, ln1b, w1t, b1, w2t, b2, ln2w, ln2b):
  full = lambda shape: pl.BlockSpec(shape, lambda i: (0, 0))
  return pl.pallas_call(
      _mlp_body,
      grid=(B // BT,),
      in_specs=[
          pl.BlockSpec((BT, EMB), lambda i: (i, 0)),
          pl.BlockSpec((BT, EMB), lambda i: (i, 0)),
          full((1, INPUT_DIM)), full((1, INPUT_DIM)),
          full((INPUT_DIM, HID)), full((1, HID)),
          full((HID, OUT)), full((1, OUT)),
          full((1, OUT)), full((1, OUT)),
      ],
      out_specs=pl.BlockSpec((BT, OUT), lambda i: (i, 0)),
      out_shape=jax.ShapeDtypeStruct((B, OUT), jnp.float32),
  )(et, el, ln1w, ln1b, w1t, b1, w2t, b2, ln2w, ln2b)


def kernel(Type, Location, emb_type, emb_loc, ln1_w, ln1_b, w1, b1, w2, b2,
           ln2_w, ln2_b):
  idx_t = jnp.reshape(Type.astype(jnp.int32), (B // CHUNK, CHUNK))
  idx_l = jnp.reshape(Location.astype(jnp.int32), (B // CHUNK, CHUNK))
  et, el = _sc_gather()(emb_type, emb_loc, idx_t, idx_l)
  return _mlp(
      et, el,
      ln1_w.reshape(1, INPUT_DIM), ln1_b.reshape(1, INPUT_DIM),
      w1.T, b1.reshape(1, HID),
      w2.T, b2.reshape(1, OUT),
      ln2_w.reshape(1, OUT), ln2_b.reshape(1, OUT),
  )
